# baseline (speedup 1.0000x reference)
import functools

import jax
import jax.numpy as jnp
from jax import lax
from jax.experimental import pallas as pl
from jax.experimental.pallas import tpu as pltpu
from jax.experimental.pallas import tpu_sc as plsc

B = 16384
E = 64
A = 18
V = 1000000
CB = 65536
VPAD = ((V + CB - 1) // CB) * CB
CHUNK = 128


def _sc_info():
    try:
        info = plsc.get_sparse_core_info()
        return info.num_cores, info.num_subcores
    except Exception:
        return 2, 16


def _tc_ptable(embed_t, W, b):
    def mk(w_ref, b_ref, e_ref, *o_refs):
        e = e_ref[...].astype(jnp.bfloat16)
        w = w_ref[...].astype(jnp.bfloat16)
        m = lax.dot_general(
            w, e,
            (((1,), (0,)), ((), ())),
            preferred_element_type=jnp.float32,
        ) + b_ref[...]
        for a in range(A):
            o_refs[a][...] = m[a:a + 1, :].reshape(CB)

    return pl.pallas_call(
        mk,
        grid=(VPAD // CB,),
        in_specs=[
            pl.BlockSpec((A, E), lambda i: (0, 0)),
            pl.BlockSpec((A, 1), lambda i: (0, 0)),
            pl.BlockSpec((E, CB), lambda i: (0, i)),
        ],
        out_specs=[pl.BlockSpec((CB,), lambda i: (i,)) for _ in range(A)],
        out_shape=[jax.ShapeDtypeStruct((VPAD,), jnp.float32) for _ in range(A)],
    )(W, b.reshape(A, 1), embed_t)


def _sc_plookup(s0, s1, s2, ptabs):
    NC, NS = _sc_info()
    NW = NC * NS
    bpw = B // NW
    nch = bpw // CHUNK
    mesh = plsc.VectorSubcoreMesh(core_axis_name="c", subcore_axis_name="s")

    @functools.partial(
        pl.kernel,
        out_type=jax.ShapeDtypeStruct((A * B,), jnp.float32),
        mesh=mesh,
        scratch_types=[
            pltpu.VMEM((bpw,), jnp.int32),
            pltpu.VMEM((bpw,), jnp.int32),
            pltpu.VMEM((bpw,), jnp.int32),
            pltpu.VMEM((nch, CHUNK), jnp.int32),
            pltpu.VMEM((A * bpw,), jnp.float32),
            pltpu.SemaphoreType.DMA,
        ],
        compiler_params=pltpu.CompilerParams(use_tc_tiling_on_sc=False),
    )
    def lookup_kernel(s0_hbm, s1_hbm, s2_hbm, *rest):
        p_hbm = rest[:A]
        out = rest[A]
        s0_v, s1_v, s2_v, ids_v, g_v, sem = rest[A + 1:]
        wid = lax.axis_index("s") * NC + lax.axis_index("c")
        base = wid * bpw
        pltpu.sync_copy(s0_hbm.at[pl.ds(base, bpw)], s0_v)
        pltpu.sync_copy(s1_hbm.at[pl.ds(base, bpw)], s1_v)
        pltpu.sync_copy(s2_hbm.at[pl.ds(base, bpw)], s2_v)
        for g in range(bpw // 16):
            sl = pl.ds(g * 16, 16)
            ids = s0_v[sl] * 10000 + s1_v[sl] * 100 + s2_v[sl]
            ids_v[(g * 16) // CHUNK, pl.ds((g * 16) % CHUNK, 16)] = ids
        for j in range(nch):
            copies = [
                pltpu.async_copy(
                    p_hbm[a].at[ids_v.at[j]],
                    g_v.at[pl.ds(a * bpw + j * CHUNK, CHUNK)],
                    sem,
                )
                for a in range(A)
            ]
            for cpy in copies:
                cpy.wait()
        for a in range(A):
            pltpu.sync_copy(
                g_v.at[pl.ds(a * bpw, bpw)],
                out.at[pl.ds(a * B + base, bpw)],
            )

    return lookup_kernel(s0, s1, s2, *ptabs)


def kernel(state, embed, W, b):
    s0, s1, s2 = state[:, 0], state[:, 1], state[:, 2]
    ptabs = _tc_ptable(embed.T, W, b)
    flat = _sc_plookup(s0, s1, s2, ptabs)
    return flat.reshape(A, B).T


# speedup vs baseline: 1.0087x; 1.0087x over previous
import functools

import jax
import jax.numpy as jnp
from jax import lax
from jax.experimental import pallas as pl
from jax.experimental.pallas import tpu as pltpu
from jax.experimental.pallas import tpu_sc as plsc

B = 16384
E = 64
A = 18
V = 1000000
CB = 32768
VPAD = ((V + CB - 1) // CB) * CB
CHUNK = 128


def _sc_info():
    try:
        info = plsc.get_sparse_core_info()
        return info.num_cores, info.num_subcores
    except Exception:
        return 2, 16


def _tc_ptable(embed_t, W, b):
    def mk(w_ref, b_ref, e_ref, *o_refs):
        e = e_ref[...].astype(jnp.bfloat16)
        w = w_ref[...].astype(jnp.bfloat16)
        m = lax.dot_general(
            w, e,
            (((1,), (0,)), ((), ())),
            preferred_element_type=jnp.float32,
        ) + b_ref[...]
        for a in range(A):
            o_refs[a][...] = m[a:a + 1, :].reshape(CB)

    return pl.pallas_call(
        mk,
        grid=(VPAD // CB,),
        in_specs=[
            pl.BlockSpec((A, E), lambda i: (0, 0)),
            pl.BlockSpec((A, 1), lambda i: (0, 0)),
            pl.BlockSpec((E, CB), lambda i: (0, i)),
        ],
        out_specs=[pl.BlockSpec((CB,), lambda i: (i,)) for _ in range(A)],
        out_shape=[jax.ShapeDtypeStruct((VPAD,), jnp.float32) for _ in range(A)],
    )(W, b.reshape(A, 1), embed_t)


def _sc_plookup(s0, s1, s2, ptabs):
    NC, NS = _sc_info()
    NW = NC * NS
    bpw = B // NW
    nch = bpw // CHUNK
    mesh = plsc.VectorSubcoreMesh(core_axis_name="c", subcore_axis_name="s")

    @functools.partial(
        pl.kernel,
        out_type=jax.ShapeDtypeStruct((A * B,), jnp.float32),
        mesh=mesh,
        scratch_types=[
            pltpu.VMEM((bpw,), jnp.int32),
            pltpu.VMEM((bpw,), jnp.int32),
            pltpu.VMEM((bpw,), jnp.int32),
            pltpu.VMEM((nch, CHUNK), jnp.int32),
            pltpu.VMEM((A * bpw,), jnp.float32),
            pltpu.SemaphoreType.DMA,
        ],
        compiler_params=pltpu.CompilerParams(use_tc_tiling_on_sc=False),
    )
    def lookup_kernel(s0_hbm, s1_hbm, s2_hbm, *rest):
        p_hbm = rest[:A]
        out = rest[A]
        s0_v, s1_v, s2_v, ids_v, g_v, sem = rest[A + 1:]
        wid = lax.axis_index("s") * NC + lax.axis_index("c")
        base = wid * bpw
        pltpu.sync_copy(s0_hbm.at[pl.ds(base, bpw)], s0_v)
        pltpu.sync_copy(s1_hbm.at[pl.ds(base, bpw)], s1_v)
        pltpu.sync_copy(s2_hbm.at[pl.ds(base, bpw)], s2_v)
        for g in range(bpw // 16):
            sl = pl.ds(g * 16, 16)
            ids = s0_v[sl] * 10000 + s1_v[sl] * 100 + s2_v[sl]
            ids_v[(g * 16) // CHUNK, pl.ds((g * 16) % CHUNK, 16)] = ids
        for j in range(nch):
            copies = [
                pltpu.async_copy(
                    p_hbm[a].at[ids_v.at[j]],
                    g_v.at[pl.ds(a * bpw + j * CHUNK, CHUNK)],
                    sem,
                )
                for a in range(A)
            ]
            for cpy in copies:
                cpy.wait()
        for a in range(A):
            pltpu.sync_copy(
                g_v.at[pl.ds(a * bpw, bpw)],
                out.at[pl.ds(a * B + base, bpw)],
            )

    return lookup_kernel(s0, s1, s2, *ptabs)


def kernel(state, embed, W, b):
    s0, s1, s2 = state[:, 0], state[:, 1], state[:, 2]
    ptabs = _tc_ptable(embed.T, W, b)
    flat = _sc_plookup(s0, s1, s2, ptabs)
    return flat.reshape(A, B).T


# speedup vs baseline: 1.0313x; 1.0224x over previous
import functools

import jax
import jax.numpy as jnp
from jax import lax
from jax.experimental import pallas as pl
from jax.experimental.pallas import tpu as pltpu
from jax.experimental.pallas import tpu_sc as plsc

B = 16384
E = 64
A = 18
V = 1000000
CB = 32768
VPAD = ((V + CB - 1) // CB) * CB
CHUNK = 128


def _sc_info():
    try:
        info = plsc.get_sparse_core_info()
        return info.num_cores, info.num_subcores
    except Exception:
        return 2, 16


def _tc_ptable(embed_t, W, b):
    def mk(w_ref, b_ref, e_ref, *o_refs):
        e = e_ref[...].astype(jnp.bfloat16)
        w = w_ref[...].astype(jnp.bfloat16)
        m = lax.dot_general(
            w, e,
            (((1,), (0,)), ((), ())),
            preferred_element_type=jnp.float32,
        ) + b_ref[...]
        for a in range(A):
            o_refs[a][...] = m[a:a + 1, :].reshape(CB)

    return pl.pallas_call(
        mk,
        grid=(VPAD // CB,),
        in_specs=[
            pl.BlockSpec((A, E), lambda i: (0, 0)),
            pl.BlockSpec((A, 1), lambda i: (0, 0)),
            pl.BlockSpec((E, CB), lambda i: (0, i)),
        ],
        out_specs=[pl.BlockSpec((CB,), lambda i: (i,)) for _ in range(A)],
        out_shape=[jax.ShapeDtypeStruct((VPAD,), jnp.float32) for _ in range(A)],
    )(W, b.reshape(A, 1), embed_t)


def _sc_plookup(s0, s1, s2, ptabs):
    NC, NS = _sc_info()
    NW = NC * NS
    bpw = B // NW
    nch = bpw // CHUNK
    mesh = plsc.VectorSubcoreMesh(core_axis_name="c", subcore_axis_name="s")

    @functools.partial(
        pl.kernel,
        out_type=jax.ShapeDtypeStruct((A * B,), jnp.float32),
        mesh=mesh,
        scratch_types=[
            pltpu.VMEM((bpw,), jnp.int32),
            pltpu.VMEM((bpw,), jnp.int32),
            pltpu.VMEM((bpw,), jnp.int32),
            pltpu.VMEM((nch, CHUNK), jnp.int32),
            pltpu.VMEM((A * bpw,), jnp.float32),
            pltpu.SemaphoreType.DMA,
        ],
        compiler_params=pltpu.CompilerParams(use_tc_tiling_on_sc=False),
    )
    def lookup_kernel(s0_hbm, s1_hbm, s2_hbm, *rest):
        p_hbm = rest[:A]
        out = rest[A]
        s0_v, s1_v, s2_v, ids_v, g_v, sem = rest[A + 1:]
        wid = lax.axis_index("s") * NC + lax.axis_index("c")
        base = wid * bpw
        pltpu.sync_copy(s0_hbm.at[pl.ds(base, bpw)], s0_v)
        pltpu.sync_copy(s1_hbm.at[pl.ds(base, bpw)], s1_v)
        pltpu.sync_copy(s2_hbm.at[pl.ds(base, bpw)], s2_v)
        for g in range(bpw // 16):
            sl = pl.ds(g * 16, 16)
            ids = s0_v[sl] * 10000 + s1_v[sl] * 100 + s2_v[sl]
            ids_v[(g * 16) // CHUNK, pl.ds((g * 16) % CHUNK, 16)] = ids
        copies = [
            pltpu.async_copy(
                p_hbm[a].at[ids_v.at[j]],
                g_v.at[pl.ds(a * bpw + j * CHUNK, CHUNK)],
                sem,
            )
            for j in range(nch)
            for a in range(A)
        ]
        for cpy in copies:
            cpy.wait()
        for a in range(A):
            pltpu.sync_copy(
                g_v.at[pl.ds(a * bpw, bpw)],
                out.at[pl.ds(a * B + base, bpw)],
            )

    return lookup_kernel(s0, s1, s2, *ptabs)


def kernel(state, embed, W, b):
    s0, s1, s2 = state[:, 0], state[:, 1], state[:, 2]
    ptabs = _tc_ptable(embed.T, W, b)
    flat = _sc_plookup(s0, s1, s2, ptabs)
    return flat.reshape(A, B).T
